# baseline (speedup 1.0000x reference)
import functools

import jax
import jax.numpy as jnp
from jax import lax
from jax.experimental import pallas as pl
from jax.experimental.pallas import tpu as pltpu
from jax.experimental.pallas import tpu_sc as plsc

B = 32
NV = 5023
NVP = 5120
CIN = 64
COUT = 128
KCHEB = 6
NVOUT = 1256
G = 8
BPG = 4
CB = BPG * CIN
BN = 1280
BIGK = 1 << 30

TROWS = NVP // 16
ETC = 2560
ECH = 64
NCHP = ETC // ECH
ACR = TROWS + 8

DRNG = NVP // 32
ETCD = 1536
DECH = 128
DACR = DRNG + 16


def _mesh():
  return plsc.VectorSubcoreMesh(core_axis_name="c", subcore_axis_name="s",
                                num_cores=2, num_subcores=16)


def _bucketize(key, payload, nbuck, rng, cap):
  n = key.shape[0]
  order = jnp.argsort(key, stable=True)
  ks = key[order]
  ps = payload[order]
  bounds = jnp.searchsorted(ks, jnp.arange(nbuck + 1, dtype=jnp.int32) * rng)
  owner = jnp.clip(ks // rng, 0, nbuck - 1)
  pos = jnp.arange(n, dtype=jnp.int32) - bounds[owner].astype(jnp.int32)
  keep = (ks < nbuck * rng) & (pos < cap)
  slot = jnp.where(keep, owner * cap + pos, nbuck * cap)
  gall = jnp.zeros((nbuck * cap + 1,), jnp.int32).at[slot].set(
      ps.astype(jnp.int32), mode="drop")
  lall = jnp.full((nbuck * cap + 1,), rng, jnp.int32).at[slot].set(
      (ks - owner * rng).astype(jnp.int32), mode="drop")
  cnts = jnp.minimum((bounds[1:] - bounds[:-1]).astype(jnp.int32), cap)
  return gall[:-1], lall[:-1], cnts


def _deg_body(ldx_hbm, cnt_hbm, deg_out, ldv, cv, accum):
  c = lax.axis_index("c")
  s = lax.axis_index("s")
  t = c * 16 + s
  zv = jnp.zeros((16,), jnp.float32)
  onesv = jnp.ones((16,), jnp.float32)

  pltpu.sync_copy(cnt_hbm, cv)
  cnt = cv[pl.ds(t, 1)][0]
  nch = (cnt + DECH - 1) // DECH

  def zrow(r, carry):
    accum[r, pl.ds(0, 16)] = zv
    return carry
  lax.fori_loop(0, DACR, zrow, 0)

  def chunk(i, carry):
    pltpu.sync_copy(ldx_hbm.at[pl.ds(t * ETCD + i * DECH, DECH)], ldv)

    def edge(e, c2):
      ld = ldv[pl.ds(e, 1)][0]
      sl = pl.ds(0, 16)
      accum[ld, sl] = accum[ld, sl] + onesv
      return c2
    lax.fori_loop(0, DECH, edge, 0)
    return carry
  lax.fori_loop(0, nch, chunk, 0)

  pltpu.sync_copy(accum.at[pl.ds(0, DRNG)],
                  deg_out.at[pl.ds(t * DRNG, DRNG)])


@functools.lru_cache(maxsize=None)
def _deg_kernel_built():
  return pl.kernel(
      _deg_body,
      out_type=jax.ShapeDtypeStruct((NVP, 16), jnp.float32),
      mesh=_mesh(),
      scratch_types=[
          pltpu.VMEM((DECH,), jnp.int32),
          pltpu.VMEM((32,), jnp.int32),
          pltpu.VMEM((DACR, 16), jnp.float32),
      ],
  )


def _deg_kernel(*args):
  return _deg_kernel_built()(*args)


def _prop_body(tbl_hbm, gidx_hbm, ldx_hbm, cnt_hbm, out_hbm, rows0, rows1,
               giv0, giv1, ldv0, ldv1, cv, accum, sem0, sem1):
  c = lax.axis_index("c")
  s = lax.axis_index("s")
  zv = jnp.zeros((16,), jnp.float32)

  pltpu.sync_copy(cnt_hbm, cv)
  cnt = cv[pl.ds(s, 1)][0]
  npair = jnp.maximum((cnt + 2 * ECH - 1) // (2 * ECH), 1)

  def issue(i, gbase, giv, ldv, rows, sem):
    base = s * ETC + i * ECH
    pltpu.sync_copy(gidx_hbm.at[pl.ds(base, ECH)], giv)
    pltpu.sync_copy(ldx_hbm.at[pl.ds(base, ECH)], ldv)
    for v in range(ECH // 16):
      sl = pl.ds(v * 16, 16)
      giv[sl] = giv[sl] + gbase
    pltpu.async_copy(tbl_hbm.at[giv], rows, sem)

  def drain_accum(giv, ldv, rows, sem):
    pltpu.make_async_copy(tbl_hbm.at[giv], rows, sem).wait()

    def edge(e, c3):
      ld = ldv[pl.ds(e, 1)][0]
      for k in range(CB // 16):
        sl = pl.ds(k * 16, 16)
        accum[ld, sl] = accum[ld, sl] + rows[e, sl]
      return c3
    lax.fori_loop(0, ECH, edge, 0)

  def group(gg, carry):
    g = c * (G // 2) + gg
    gbase = jnp.zeros((16,), jnp.int32) + g * NVP

    def zrow(r, c2):
      for k in range(CB // 16):
        accum[r, pl.ds(k * 16, 16)] = zv
      return c2
    lax.fori_loop(0, TROWS, zrow, 0)

    issue(0, gbase, giv0, ldv0, rows0, sem0)

    def pair(k, c2):
      issue(2 * k + 1, gbase, giv1, ldv1, rows1, sem1)
      drain_accum(giv0, ldv0, rows0, sem0)
      issue(2 * k + 2, gbase, giv0, ldv0, rows0, sem0)
      drain_accum(giv1, ldv1, rows1, sem1)
      return c2
    lax.fori_loop(0, npair - 1, pair, 0)

    issue(2 * npair - 1, gbase, giv1, ldv1, rows1, sem1)
    drain_accum(giv0, ldv0, rows0, sem0)
    drain_accum(giv1, ldv1, rows1, sem1)

    pltpu.sync_copy(accum.at[pl.ds(0, TROWS)],
                    out_hbm.at[pl.ds(g * NVP + s * TROWS, TROWS)])
    return carry
  lax.fori_loop(0, G // 2, group, 0)


@functools.lru_cache(maxsize=None)
def _prop_kernel_built():
  return pl.kernel(
      _prop_body,
      out_type=jax.ShapeDtypeStruct((G * NVP, CB), jnp.float32),
      mesh=_mesh(),
      scratch_types=[
          pltpu.VMEM((ECH, CB), jnp.float32),
          pltpu.VMEM((ECH, CB), jnp.float32),
          pltpu.VMEM((ECH,), jnp.int32),
          pltpu.VMEM((ECH,), jnp.int32),
          pltpu.VMEM((ECH,), jnp.int32),
          pltpu.VMEM((ECH,), jnp.int32),
          pltpu.VMEM((16,), jnp.int32),
          pltpu.VMEM((ACR, CB), jnp.float32),
          pltpu.SemaphoreType.DMA,
          pltpu.SemaphoreType.DMA,
      ],
  )


def _prop_kernel(*args):
  return _prop_kernel_built()(*args)


def _dinv_from(deg_ref):
  deg = deg_ref[:, 0]
  return jnp.where(deg > 0, lax.rsqrt(deg), 0.0)


def _round0_body(xt_ref, deg_ref, w0_ref, d0_ref, acc_ref):
  dinv = _dinv_from(deg_ref)
  xb = xt_ref[0]
  d0_ref[0] = xb * dinv[:, None]
  for l in range(BPG):
    acc_ref[0, l] = jnp.dot(xb[:, l * CIN:(l + 1) * CIN], w0_ref[...],
                            preferred_element_type=jnp.float32)


def _tc_round0(xt, deg2d, w0):
  return pl.pallas_call(
      _round0_body,
      grid=(G, NVP // BN),
      in_specs=[
          pl.BlockSpec((1, BN, CB), lambda g, i: (g, i, 0)),
          pl.BlockSpec((BN, 16), lambda g, i: (i, 0)),
          pl.BlockSpec((CIN, COUT), lambda g, i: (0, 0)),
      ],
      out_specs=[
          pl.BlockSpec((1, BN, CB), lambda g, i: (g, i, 0)),
          pl.BlockSpec((1, BPG, BN, COUT), lambda g, i: (g, 0, i, 0)),
      ],
      out_shape=[
          jax.ShapeDtypeStruct((G, NVP, CB), jnp.float32),
          jax.ShapeDtypeStruct((G, BPG, NVP, COUT), jnp.float32),
      ],
  )(xt, deg2d, w0)


def _roundj_body(r_ref, txm2_ref, deg_ref, wj_ref, accin_ref,
                 txj_ref, dj_ref, accout_ref, *, cj, want_d):
  dinv = _dinv_from(deg_ref)
  txj = (-cj) * (dinv[:, None] * r_ref[0])
  if txm2_ref is not None:
    txj = txj - txm2_ref[0]
  txj_ref[0] = txj
  if want_d:
    dj_ref[0] = txj * dinv[:, None]
  for l in range(BPG):
    accout_ref[0, l] = accin_ref[0, l] + jnp.dot(
        txj[:, l * CIN:(l + 1) * CIN], wj_ref[...],
        preferred_element_type=jnp.float32)


def _tc_roundj(r, txm2, deg2d, wj, acc, cj, want_d):
  tbl = pl.BlockSpec((1, BN, CB), lambda g, i: (g, i, 0))
  degs = pl.BlockSpec((BN, 16), lambda g, i: (i, 0))
  ws = pl.BlockSpec((CIN, COUT), lambda g, i: (0, 0))
  accs = pl.BlockSpec((1, BPG, BN, COUT), lambda g, i: (g, 0, i, 0))
  have_t = txm2 is not None
  if have_t:
    body = functools.partial(_roundj_body, cj=cj, want_d=want_d)
    ins = (r, txm2, deg2d, wj, acc)
    in_specs = [tbl, tbl, degs, ws, accs]
    alias_in = 4
  else:
    body = functools.partial(
        lambda r_ref, deg_ref, wj_ref, accin_ref, *outs, cj, want_d:
        _roundj_body(r_ref, None, deg_ref, wj_ref, accin_ref, *outs,
                     cj=cj, want_d=want_d), cj=cj, want_d=want_d)
    ins = (r, deg2d, wj, acc)
    in_specs = [tbl, degs, ws, accs]
    alias_in = 3
  out_specs = [tbl]
  out_shape = [jax.ShapeDtypeStruct((G, NVP, CB), jnp.float32)]
  if want_d:
    out_specs.append(tbl)
    out_shape.append(jax.ShapeDtypeStruct((G, NVP, CB), jnp.float32))
  else:
    body = functools.partial(_insert_none_dj, body)
  out_specs.append(accs)
  out_shape.append(jax.ShapeDtypeStruct((G, BPG, NVP, COUT), jnp.float32))
  acc_out_idx = len(out_specs) - 1
  return pl.pallas_call(
      body,
      grid=(G, NVP // BN),
      in_specs=in_specs,
      out_specs=out_specs,
      out_shape=out_shape,
      input_output_aliases={alias_in: acc_out_idx},
  )(*ins)


def _insert_none_dj(body, *refs):
  body(*refs[:-2], refs[-2], None, refs[-1])


def _final_body(y_ref, b_ref, s_ref, out_ref):
  z = y_ref[0] + b_ref[...][None, :]
  z = jnp.where(z > 0, z, jnp.exp(z) - 1.0)
  out_ref[0] = lax.dot_general(s_ref[...], z, (((1,), (0,)), ((), ())),
                               preferred_element_type=jnp.float32)


def _tc_final(y32, bias, s_pad):
  return pl.pallas_call(
      _final_body,
      grid=(B,),
      in_specs=[
          pl.BlockSpec((1, NVP, COUT), lambda i: (i, 0, 0)),
          pl.BlockSpec((COUT,), lambda i: (0,)),
          pl.BlockSpec((NVOUT, NVP), lambda i: (0, 0)),
      ],
      out_specs=pl.BlockSpec((1, NVOUT, COUT), lambda i: (i, 0, 0)),
      out_shape=jax.ShapeDtypeStruct((B, NVOUT, COUT), jnp.float32),
  )(y32, bias, s_pad)


def kernel(x, edge_index, W, b, S):
  src = edge_index[0].astype(jnp.int32)
  dst = edge_index[1].astype(jnp.int32)
  loop = src == dst
  dkey = jnp.where(loop, BIGK, dst)
  skey = jnp.where(loop, BIGK, src)
  gidx, ldx, cntp = _bucketize(dkey, src, 16, TROWS, ETC)
  _, ldx_deg, cntd = _bucketize(skey, src, 32, DRNG, ETCD)

  def _ilv(a, ch, st):
    return a.reshape(-1, st, ch // st).transpose(0, 2, 1).reshape(-1)
  gidx = _ilv(gidx, ECH, 8)
  ldx = _ilv(ldx, ECH, 8)
  ldx_deg = _ilv(ldx_deg, DECH, 16)

  xt = x.reshape(G, BPG, NV, CIN).transpose(0, 2, 1, 3).reshape(G, NV, CB)
  xt = jnp.pad(xt, ((0, 0), (0, NVP - NV), (0, 0)))
  s_pad = jnp.pad(S, ((0, 0), (0, NVP - NV)))

  deg2d = _deg_kernel(ldx_deg, cntd)

  d_prev, acc = _tc_round0(xt, deg2d, W[0])
  tx_m2 = xt
  tx_m1 = None
  for j in range(1, KCHEB):
    r = _prop_kernel(d_prev.reshape(G * NVP, CB), gidx, ldx, cntp)
    r = r.reshape(G, NVP, CB)
    want_d = j < KCHEB - 1
    outs = _tc_roundj(r, tx_m2 if j >= 2 else None, deg2d, W[j], acc,
                      cj=2.0 if j >= 2 else 1.0, want_d=want_d)
    if want_d:
      txj, dj, acc = outs
      d_prev = dj
    else:
      txj, acc = outs
    tx_m2, tx_m1 = tx_m1, txj
    if j == 1:
      tx_m2 = xt

  y32 = acc.reshape(B, NVP, COUT)
  return _tc_final(y32, b, s_pad)


# speedup vs baseline: 1.0110x; 1.0110x over previous
import functools

import jax
import jax.numpy as jnp
from jax import lax
from jax.experimental import pallas as pl
from jax.experimental.pallas import tpu as pltpu
from jax.experimental.pallas import tpu_sc as plsc

B = 32
NV = 5023
NVP = 5120
CIN = 64
COUT = 128
KCHEB = 6
NVOUT = 1256
G = 8
BPG = 4
CB = BPG * CIN
BN = 1280
BIGK = 1 << 30

TROWS = NVP // 16
ETC = 2560
ECH = 64
NCHP = ETC // ECH
ACR = TROWS + 8

DRNG = NVP // 32
ETCD = 1536
DECH = 128
DACR = DRNG + 16


def _mesh():
  return plsc.VectorSubcoreMesh(core_axis_name="c", subcore_axis_name="s",
                                num_cores=2, num_subcores=16)


def _bucketize(key, payload, nbuck, rng, cap):
  n = key.shape[0]
  order = jnp.argsort(key, stable=True)
  ks = key[order]
  ps = payload[order]
  bounds = jnp.searchsorted(ks, jnp.arange(nbuck + 1, dtype=jnp.int32) * rng)
  owner = jnp.clip(ks // rng, 0, nbuck - 1)
  pos = jnp.arange(n, dtype=jnp.int32) - bounds[owner].astype(jnp.int32)
  keep = (ks < nbuck * rng) & (pos < cap)
  slot = jnp.where(keep, owner * cap + pos, nbuck * cap)
  gall = jnp.zeros((nbuck * cap + 1,), jnp.int32).at[slot].set(
      ps.astype(jnp.int32), mode="drop")
  lall = jnp.full((nbuck * cap + 1,), rng, jnp.int32).at[slot].set(
      (ks - owner * rng).astype(jnp.int32), mode="drop")
  cnts = jnp.minimum((bounds[1:] - bounds[:-1]).astype(jnp.int32), cap)
  return gall[:-1], lall[:-1], cnts


def _deg_body(ldx_hbm, cnt_hbm, deg_out, ldv, cv, accum):
  c = lax.axis_index("c")
  s = lax.axis_index("s")
  t = c * 16 + s
  zv = jnp.zeros((16,), jnp.float32)
  onesv = jnp.ones((16,), jnp.float32)

  pltpu.sync_copy(cnt_hbm, cv)
  cnt = cv[pl.ds(t, 1)][0]
  nch = (cnt + DECH - 1) // DECH

  def zrow(r, carry):
    accum[r, pl.ds(0, 16)] = zv
    return carry
  lax.fori_loop(0, DACR, zrow, 0)

  def chunk(i, carry):
    pltpu.sync_copy(ldx_hbm.at[pl.ds(t * ETCD + i * DECH, DECH)], ldv)

    def edge(e, c2):
      ld = ldv[pl.ds(e, 1)][0]
      sl = pl.ds(0, 16)
      accum[ld, sl] = accum[ld, sl] + onesv
      return c2
    lax.fori_loop(0, DECH, edge, 0)
    return carry
  lax.fori_loop(0, nch, chunk, 0)

  pltpu.sync_copy(accum.at[pl.ds(0, DRNG)],
                  deg_out.at[pl.ds(t * DRNG, DRNG)])


@functools.lru_cache(maxsize=None)
def _deg_kernel_built():
  return pl.kernel(
      _deg_body,
      out_type=jax.ShapeDtypeStruct((NVP, 16), jnp.float32),
      mesh=_mesh(),
      scratch_types=[
          pltpu.VMEM((DECH,), jnp.int32),
          pltpu.VMEM((32,), jnp.int32),
          pltpu.VMEM((DACR, 16), jnp.float32),
      ],
  )


def _deg_kernel(*args):
  return _deg_kernel_built()(*args)


def _prop_body(tbl_hbm, gidx_hbm, ldx_hbm, cnt_hbm, out_hbm, rows0, rows1,
               giv0, giv1, ldv0, ldv1, cv, accum, sem0, sem1):
  c = lax.axis_index("c")
  s = lax.axis_index("s")
  zv = jnp.zeros((16,), jnp.float32)

  pltpu.sync_copy(cnt_hbm, cv)
  cnt = cv[pl.ds(s, 1)][0]
  npair = jnp.maximum((cnt + 2 * ECH - 1) // (2 * ECH), 1)

  def issue(i, gbase, giv, ldv, rows, sem):
    base = s * ETC + i * ECH
    pltpu.sync_copy(gidx_hbm.at[pl.ds(base, ECH)], giv)
    pltpu.sync_copy(ldx_hbm.at[pl.ds(base, ECH)], ldv)
    for v in range(ECH // 16):
      sl = pl.ds(v * 16, 16)
      giv[sl] = giv[sl] + gbase
    pltpu.async_copy(tbl_hbm.at[giv], rows, sem)

  def drain_accum(giv, ldv, rows, sem):
    pltpu.make_async_copy(tbl_hbm.at[giv], rows, sem).wait()

    def edge(e, c3):
      ld = ldv[pl.ds(e, 1)][0]
      for k in range(CB // 16):
        sl = pl.ds(k * 16, 16)
        accum[ld, sl] = accum[ld, sl] + rows[e, sl]
      return c3
    lax.fori_loop(0, ECH, edge, 0)

  def group(gg, carry):
    g = c * (G // 2) + gg
    gbase = jnp.zeros((16,), jnp.int32) + g * NVP

    def zrow(r, c2):
      for k in range(CB // 16):
        accum[r, pl.ds(k * 16, 16)] = zv
      return c2
    lax.fori_loop(0, TROWS, zrow, 0)

    issue(0, gbase, giv0, ldv0, rows0, sem0)

    def pair(k, c2):
      issue(2 * k + 1, gbase, giv1, ldv1, rows1, sem1)
      drain_accum(giv0, ldv0, rows0, sem0)
      issue(2 * k + 2, gbase, giv0, ldv0, rows0, sem0)
      drain_accum(giv1, ldv1, rows1, sem1)
      return c2
    lax.fori_loop(0, npair - 1, pair, 0)

    issue(2 * npair - 1, gbase, giv1, ldv1, rows1, sem1)
    drain_accum(giv0, ldv0, rows0, sem0)
    drain_accum(giv1, ldv1, rows1, sem1)

    pltpu.sync_copy(accum.at[pl.ds(0, TROWS)],
                    out_hbm.at[pl.ds(g * NVP + s * TROWS, TROWS)])
    return carry
  lax.fori_loop(0, G // 2, group, 0)


@functools.lru_cache(maxsize=None)
def _prop_kernel_built():
  return pl.kernel(
      _prop_body,
      out_type=jax.ShapeDtypeStruct((G * NVP, CB), jnp.float32),
      mesh=_mesh(),
      scratch_types=[
          pltpu.VMEM((ECH, CB), jnp.float32),
          pltpu.VMEM((ECH, CB), jnp.float32),
          pltpu.VMEM((ECH,), jnp.int32),
          pltpu.VMEM((ECH,), jnp.int32),
          pltpu.VMEM((ECH,), jnp.int32),
          pltpu.VMEM((ECH,), jnp.int32),
          pltpu.VMEM((16,), jnp.int32),
          pltpu.VMEM((ACR, CB), jnp.float32),
          pltpu.SemaphoreType.DMA,
          pltpu.SemaphoreType.DMA,
      ],
  )


def _prop_kernel(*args):
  return _prop_kernel_built()(*args)


def _dinv_from(deg_ref):
  deg = deg_ref[:, 0]
  return jnp.where(deg > 0, lax.rsqrt(deg), 0.0)


def _round0_body(xt_ref, deg_ref, w0_ref, d0_ref, acc_ref):
  dinv = _dinv_from(deg_ref)
  xb = xt_ref[0]
  d0_ref[0] = xb * dinv[:, None]
  for l in range(BPG):
    acc_ref[0, l] = jnp.dot(xb[:, l * CIN:(l + 1) * CIN], w0_ref[...],
                            preferred_element_type=jnp.float32)


def _tc_round0(xt, deg2d, w0):
  return pl.pallas_call(
      _round0_body,
      grid=(G, NVP // BN),
      in_specs=[
          pl.BlockSpec((1, BN, CB), lambda g, i: (g, i, 0)),
          pl.BlockSpec((BN, 16), lambda g, i: (i, 0)),
          pl.BlockSpec((CIN, COUT), lambda g, i: (0, 0)),
      ],
      out_specs=[
          pl.BlockSpec((1, BN, CB), lambda g, i: (g, i, 0)),
          pl.BlockSpec((1, BPG, BN, COUT), lambda g, i: (g, 0, i, 0)),
      ],
      out_shape=[
          jax.ShapeDtypeStruct((G, NVP, CB), jnp.float32),
          jax.ShapeDtypeStruct((G, BPG, NVP, COUT), jnp.float32),
      ],
  )(xt, deg2d, w0)


def _roundj_body(r_ref, txm2_ref, deg_ref, wj_ref, accin_ref,
                 txj_ref, dj_ref, accout_ref, *, cj, want_d):
  dinv = _dinv_from(deg_ref)
  txj = (-cj) * (dinv[:, None] * r_ref[0])
  if txm2_ref is not None:
    txj = txj - txm2_ref[0]
  txj_ref[0] = txj
  if want_d:
    dj_ref[0] = txj * dinv[:, None]
  for l in range(BPG):
    accout_ref[0, l] = accin_ref[0, l] + jnp.dot(
        txj[:, l * CIN:(l + 1) * CIN], wj_ref[...],
        preferred_element_type=jnp.float32)


def _tc_roundj(r, txm2, deg2d, wj, acc, cj, want_d):
  tbl = pl.BlockSpec((1, BN, CB), lambda g, i: (g, i, 0))
  degs = pl.BlockSpec((BN, 16), lambda g, i: (i, 0))
  ws = pl.BlockSpec((CIN, COUT), lambda g, i: (0, 0))
  accs = pl.BlockSpec((1, BPG, BN, COUT), lambda g, i: (g, 0, i, 0))
  have_t = txm2 is not None
  if have_t:
    body = functools.partial(_roundj_body, cj=cj, want_d=want_d)
    ins = (r, txm2, deg2d, wj, acc)
    in_specs = [tbl, tbl, degs, ws, accs]
    alias_in = 4
  else:
    body = functools.partial(
        lambda r_ref, deg_ref, wj_ref, accin_ref, *outs, cj, want_d:
        _roundj_body(r_ref, None, deg_ref, wj_ref, accin_ref, *outs,
                     cj=cj, want_d=want_d), cj=cj, want_d=want_d)
    ins = (r, deg2d, wj, acc)
    in_specs = [tbl, degs, ws, accs]
    alias_in = 3
  out_specs = [tbl]
  out_shape = [jax.ShapeDtypeStruct((G, NVP, CB), jnp.float32)]
  if want_d:
    out_specs.append(tbl)
    out_shape.append(jax.ShapeDtypeStruct((G, NVP, CB), jnp.float32))
  else:
    body = functools.partial(_insert_none_dj, body)
  out_specs.append(accs)
  out_shape.append(jax.ShapeDtypeStruct((G, BPG, NVP, COUT), jnp.float32))
  acc_out_idx = len(out_specs) - 1
  return pl.pallas_call(
      body,
      grid=(G, NVP // BN),
      in_specs=in_specs,
      out_specs=out_specs,
      out_shape=out_shape,
      input_output_aliases={alias_in: acc_out_idx},
  )(*ins)


def _insert_none_dj(body, *refs):
  body(*refs[:-2], refs[-2], None, refs[-1])


def _final_body(y_ref, b_ref, s_ref, out_ref):
  z = y_ref[0] + b_ref[...][None, :]
  z = jnp.where(z > 0, z, jnp.exp(z) - 1.0)
  out_ref[0] = lax.dot_general(s_ref[...], z, (((1,), (0,)), ((), ())),
                               preferred_element_type=jnp.float32)


def _tc_final(y32, bias, s_pad):
  return pl.pallas_call(
      _final_body,
      grid=(B,),
      in_specs=[
          pl.BlockSpec((1, NVP, COUT), lambda i: (i, 0, 0)),
          pl.BlockSpec((COUT,), lambda i: (0,)),
          pl.BlockSpec((NVOUT, NVP), lambda i: (0, 0)),
      ],
      out_specs=pl.BlockSpec((1, NVOUT, COUT), lambda i: (i, 0, 0)),
      out_shape=jax.ShapeDtypeStruct((B, NVOUT, COUT), jnp.float32),
  )(y32, bias, s_pad)


def kernel(x, edge_index, W, b, S):
  src = edge_index[0].astype(jnp.int32)
  dst = edge_index[1].astype(jnp.int32)
  loop = src == dst
  dkey = jnp.where(loop, BIGK, dst)
  skey = jnp.where(loop, BIGK, src)
  gidx, ldx, cntp = _bucketize(dkey, src, 16, TROWS, ETC)
  _, ldx_deg, cntd = _bucketize(skey, src, 32, DRNG, ETCD)

  xt = x.reshape(G, BPG, NV, CIN).transpose(0, 2, 1, 3).reshape(G, NV, CB)
  xt = jnp.pad(xt, ((0, 0), (0, NVP - NV), (0, 0)))
  s_pad = jnp.pad(S, ((0, 0), (0, NVP - NV)))

  deg2d = _deg_kernel(ldx_deg, cntd)

  d_prev, acc = _tc_round0(xt, deg2d, W[0])
  tx_m2 = xt
  tx_m1 = None
  for j in range(1, KCHEB):
    r = _prop_kernel(d_prev.reshape(G * NVP, CB), gidx, ldx, cntp)
    r = r.reshape(G, NVP, CB)
    want_d = j < KCHEB - 1
    outs = _tc_roundj(r, tx_m2 if j >= 2 else None, deg2d, W[j], acc,
                      cj=2.0 if j >= 2 else 1.0, want_d=want_d)
    if want_d:
      txj, dj, acc = outs
      d_prev = dj
    else:
      txj, acc = outs
    tx_m2, tx_m1 = tx_m1, txj
    if j == 1:
      tx_m2 = xt

  y32 = acc.reshape(B, NVP, COUT)
  return _tc_final(y32, b, s_pad)
